# Initial kernel scaffold; baseline (speedup 1.0000x reference)
import jax
import jax.numpy as jnp
from jax.experimental import pallas as pl


def kernel(decoder_input_ids, encoder_hidden_states, encoder_input_mask, emb_table, pos_emb, W_dec, W_enc):
    raise NotImplementedError("write your pallas kernel here")


# speedup vs baseline: 2.4958x; 2.4958x over previous
import functools

import jax
import jax.numpy as jnp
from jax.experimental import pallas as pl

V = 100000
D = 512
B = 16
BEAM = 4
STEPS = 6
PAD, BOS, EOS = 0, 1, 2
LEN_PEN = 0.6
NEG_INF = -1e9

VC = 2000
NC = V // VC
LANES = 128


def _chunk_kernel(iota_ref, hT_ref, emb_ref, m_ref, s_ref, v_ref, i_ref,
                  *, n_rows):
    nsub = LANES // n_rows
    sub = VC // nsub
    tiles = [
        jax.lax.dot_general(
            emb_ref[s * sub:(s + 1) * sub, :], hT_ref[...],
            dimension_numbers=(((1,), (0,)), ((), ())),
            preferred_element_type=jnp.float32)
        for s in range(nsub)
    ]
    logits = jnp.concatenate(tiles, axis=1)

    m = jnp.max(logits, axis=0)
    s_ref[0, 0, :] = jnp.sum(jnp.exp(logits - m[None, :]), axis=0)
    m_ref[0, 0, :] = m

    row = iota_ref[...]
    vals = logits
    mt = m
    for t in range(BEAM):
        if t:
            mt = jnp.max(vals, axis=0)
        ki = jnp.where(vals == mt[None, :], row, sub)
        at = jnp.min(ki, axis=0)
        v_ref[0, t, :] = mt
        i_ref[0, t, :] = at
        if t + 1 < BEAM:
            vals = jnp.where(ki == at[None, :], NEG_INF, vals)


def _stream_pass(hT, emb_table, n_rows):
    nsub = LANES // n_rows
    sub = VC // nsub
    iota = jnp.broadcast_to(
        jnp.arange(sub, dtype=jnp.int32)[:, None], (sub, LANES))
    kern = functools.partial(_chunk_kernel, n_rows=n_rows)
    return pl.pallas_call(
        kern,
        grid=(NC,),
        in_specs=[
            pl.BlockSpec((sub, LANES), lambda j: (0, 0)),
            pl.BlockSpec((D, n_rows), lambda j: (0, 0)),
            pl.BlockSpec((VC, D), lambda j: (j, 0)),
        ],
        out_specs=[
            pl.BlockSpec((1, 1, LANES), lambda j: (j, 0, 0)),
            pl.BlockSpec((1, 1, LANES), lambda j: (j, 0, 0)),
            pl.BlockSpec((1, BEAM, LANES), lambda j: (j, 0, 0)),
            pl.BlockSpec((1, BEAM, LANES), lambda j: (j, 0, 0)),
        ],
        out_shape=[
            jax.ShapeDtypeStruct((NC, 1, LANES), jnp.float32),
            jax.ShapeDtypeStruct((NC, 1, LANES), jnp.float32),
            jax.ShapeDtypeStruct((NC, BEAM, LANES), jnp.float32),
            jax.ShapeDtypeStruct((NC, BEAM, LANES), jnp.int32),
        ],
    )(iota, hT, emb_table)


def _merge_stats(m_c, s_c, v_c, i_c, n_rows):
    nsub = LANES // n_rows
    sub = VC // nsub
    m_c = m_c.reshape(NC, 1, nsub, n_rows)[:, 0]
    s_c = s_c.reshape(NC, 1, nsub, n_rows)[:, 0]
    m2 = m_c.reshape(NC * nsub, n_rows)
    s2 = s_c.reshape(NC * nsub, n_rows)
    mx = jnp.max(m2, axis=0)
    lse = mx + jnp.log(jnp.sum(s2 * jnp.exp(m2 - mx[None, :]), axis=0))
    v4 = v_c.reshape(NC, BEAM, nsub, n_rows)
    i4 = i_c.reshape(NC, BEAM, nsub, n_rows)
    chunk_base = (jnp.arange(NC, dtype=jnp.int32) * VC)[:, None, None, None]
    sub_base = (jnp.arange(nsub, dtype=jnp.int32) * sub)[None, None, :, None]
    g4 = chunk_base + sub_base + i4
    vals = v4.transpose(0, 2, 1, 3).reshape(NC * nsub * BEAM, n_rows).T
    idx = g4.transpose(0, 2, 1, 3).reshape(NC * nsub * BEAM, n_rows).T
    top_v, pos = jax.lax.top_k(vals, BEAM)
    top_i = jnp.take_along_axis(idx, pos, axis=1)
    return top_v, top_i, lse


def kernel(decoder_input_ids, encoder_hidden_states, encoder_input_mask,
           emb_table, pos_emb, W_dec, W_enc):
    mask = encoder_input_mask
    enc_ctx = (encoder_hidden_states * mask[:, :, None]).sum(axis=1) / \
        jnp.maximum(mask.sum(axis=1, keepdims=True), 1.0)

    tok0 = decoder_input_ids[:, 0]
    h = jnp.take(emb_table, tok0, axis=0) + pos_emb[0][None, :]
    h = jnp.tanh(h @ W_dec + enc_ctx @ W_enc)
    top_v, top_i, lse = _merge_stats(*_stream_pass(h.T, emb_table, B), B)
    scores = (top_v - lse[:, None]).reshape(-1, 1)
    prefixes = jnp.concatenate(
        [jnp.repeat(decoder_input_ids, BEAM, axis=0),
         top_i.reshape(-1, 1).astype(jnp.int32)], axis=1)
    ctx_rep = jnp.repeat(enc_ctx, BEAM, axis=0)
    ctx_enc = ctx_rep @ W_enc
    pad_profile = jnp.zeros((B * BEAM,), dtype=jnp.int32)
    n = B * BEAM

    for i in range(1, STEPS + 1):
        tok = prefixes[:, -1]
        h = jnp.take(emb_table, tok, axis=0) + pos_emb[i][None, :]
        h = jnp.tanh(h @ W_dec + ctx_enc)
        top_v, top_i, lse = _merge_stats(*_stream_pass(h.T, emb_table, n), n)
        lp_cand = top_v - lse[:, None]
        finished = pad_profile > 0
        pad_lp = jnp.where(jnp.arange(BEAM) == 0, 0.0, NEG_INF)
        lp_cand = jnp.where(finished[:, None], pad_lp[None, :], lp_cand)
        tok_cand = jnp.where(finished[:, None], PAD, top_i)

        total = scores + lp_cand
        length = prefixes.shape[1]
        penalty = ((5.0 + length) / 6.0) ** LEN_PEN
        cand = (total / penalty).reshape(B, BEAM * BEAM)
        _, tpos = jax.lax.top_k(cand, BEAM)
        beam_idx = tpos // BEAM
        token = jnp.take_along_axis(
            tok_cand.reshape(B, BEAM * BEAM), tpos, axis=1).astype(jnp.int32)
        new_scores = jnp.take_along_axis(
            total.reshape(B, BEAM * BEAM), tpos, axis=1).reshape(-1, 1)
        flat_beam = (beam_idx + jnp.arange(B)[:, None] * BEAM).reshape(-1)
        prefixes = jnp.concatenate(
            [prefixes[flat_beam], token.reshape(-1, 1)], axis=1)
        scores = new_scores
        pad_profile = jnp.maximum(
            pad_profile[flat_beam], (token.reshape(-1) == EOS).astype(jnp.int32))

    return prefixes, scores.reshape(B, BEAM)


# speedup vs baseline: 2.8868x; 1.1566x over previous
import functools

import jax
import jax.numpy as jnp
from jax.experimental import pallas as pl

V = 100000
D = 512
B = 16
BEAM = 4
STEPS = 6
PAD, BOS, EOS = 0, 1, 2
LEN_PEN = 0.6
NEG_INF = -1e9

VC = 4000
NC = V // VC
LANES = 128


def _chunk_kernel(iota_ref, hT_ref, emb_ref, m_ref, s_ref, v_ref, i_ref,
                  *, n_rows):
    nsub = LANES // n_rows
    sub = VC // nsub
    tiles = [
        jax.lax.dot_general(
            emb_ref[s * sub:(s + 1) * sub, :], hT_ref[...],
            dimension_numbers=(((1,), (0,)), ((), ())),
            preferred_element_type=jnp.float32)
        for s in range(nsub)
    ]
    logits = jnp.concatenate(tiles, axis=1)

    m = jnp.max(logits, axis=0)
    s_ref[0, 0, :] = jnp.sum(jnp.exp(logits - m[None, :]), axis=0)
    m_ref[0, 0, :] = m

    row = iota_ref[...]
    vals = logits
    mt = m
    for t in range(BEAM):
        if t:
            mt = jnp.max(vals, axis=0)
        ki = jnp.where(vals == mt[None, :], row, sub)
        at = jnp.min(ki, axis=0)
        v_ref[0, t, :] = mt
        i_ref[0, t, :] = at
        if t + 1 < BEAM:
            vals = jnp.where(ki == at[None, :], NEG_INF, vals)


def _stream_pass(hT, emb_table, n_rows):
    nsub = LANES // n_rows
    sub = VC // nsub
    iota = jnp.broadcast_to(
        jnp.arange(sub, dtype=jnp.int32)[:, None], (sub, LANES))
    kern = functools.partial(_chunk_kernel, n_rows=n_rows)
    return pl.pallas_call(
        kern,
        grid=(NC,),
        in_specs=[
            pl.BlockSpec((sub, LANES), lambda j: (0, 0)),
            pl.BlockSpec((D, n_rows), lambda j: (0, 0)),
            pl.BlockSpec((VC, D), lambda j: (j, 0)),
        ],
        out_specs=[
            pl.BlockSpec((1, 1, LANES), lambda j: (j, 0, 0)),
            pl.BlockSpec((1, 1, LANES), lambda j: (j, 0, 0)),
            pl.BlockSpec((1, BEAM, LANES), lambda j: (j, 0, 0)),
            pl.BlockSpec((1, BEAM, LANES), lambda j: (j, 0, 0)),
        ],
        out_shape=[
            jax.ShapeDtypeStruct((NC, 1, LANES), jnp.float32),
            jax.ShapeDtypeStruct((NC, 1, LANES), jnp.float32),
            jax.ShapeDtypeStruct((NC, BEAM, LANES), jnp.float32),
            jax.ShapeDtypeStruct((NC, BEAM, LANES), jnp.int32),
        ],
    )(iota, hT, emb_table)


def _merge_stats(m_c, s_c, v_c, i_c, n_rows):
    nsub = LANES // n_rows
    sub = VC // nsub
    m_c = m_c.reshape(NC, 1, nsub, n_rows)[:, 0]
    s_c = s_c.reshape(NC, 1, nsub, n_rows)[:, 0]
    m2 = m_c.reshape(NC * nsub, n_rows)
    s2 = s_c.reshape(NC * nsub, n_rows)
    mx = jnp.max(m2, axis=0)
    lse = mx + jnp.log(jnp.sum(s2 * jnp.exp(m2 - mx[None, :]), axis=0))
    v4 = v_c.reshape(NC, BEAM, nsub, n_rows)
    i4 = i_c.reshape(NC, BEAM, nsub, n_rows)
    chunk_base = (jnp.arange(NC, dtype=jnp.int32) * VC)[:, None, None, None]
    sub_base = (jnp.arange(nsub, dtype=jnp.int32) * sub)[None, None, :, None]
    g4 = chunk_base + sub_base + i4
    vals = v4.transpose(0, 2, 1, 3).reshape(NC * nsub * BEAM, n_rows).T
    idx = g4.transpose(0, 2, 1, 3).reshape(NC * nsub * BEAM, n_rows).T
    top_v, pos = jax.lax.top_k(vals, BEAM)
    top_i = jnp.take_along_axis(idx, pos, axis=1)
    return top_v, top_i, lse


def kernel(decoder_input_ids, encoder_hidden_states, encoder_input_mask,
           emb_table, pos_emb, W_dec, W_enc):
    mask = encoder_input_mask
    enc_ctx = (encoder_hidden_states * mask[:, :, None]).sum(axis=1) / \
        jnp.maximum(mask.sum(axis=1, keepdims=True), 1.0)

    tok0 = decoder_input_ids[:, 0]
    h = jnp.take(emb_table, tok0, axis=0) + pos_emb[0][None, :]
    h = jnp.tanh(h @ W_dec + enc_ctx @ W_enc)
    top_v, top_i, lse = _merge_stats(*_stream_pass(h.T, emb_table, B), B)
    scores = (top_v - lse[:, None]).reshape(-1, 1)
    prefixes = jnp.concatenate(
        [jnp.repeat(decoder_input_ids, BEAM, axis=0),
         top_i.reshape(-1, 1).astype(jnp.int32)], axis=1)
    ctx_rep = jnp.repeat(enc_ctx, BEAM, axis=0)
    ctx_enc = ctx_rep @ W_enc
    pad_profile = jnp.zeros((B * BEAM,), dtype=jnp.int32)
    n = B * BEAM

    for i in range(1, STEPS + 1):
        tok = prefixes[:, -1]
        h = jnp.take(emb_table, tok, axis=0) + pos_emb[i][None, :]
        h = jnp.tanh(h @ W_dec + ctx_enc)
        top_v, top_i, lse = _merge_stats(*_stream_pass(h.T, emb_table, n), n)
        lp_cand = top_v - lse[:, None]
        finished = pad_profile > 0
        pad_lp = jnp.where(jnp.arange(BEAM) == 0, 0.0, NEG_INF)
        lp_cand = jnp.where(finished[:, None], pad_lp[None, :], lp_cand)
        tok_cand = jnp.where(finished[:, None], PAD, top_i)

        total = scores + lp_cand
        length = prefixes.shape[1]
        penalty = ((5.0 + length) / 6.0) ** LEN_PEN
        cand = (total / penalty).reshape(B, BEAM * BEAM)
        _, tpos = jax.lax.top_k(cand, BEAM)
        beam_idx = tpos // BEAM
        token = jnp.take_along_axis(
            tok_cand.reshape(B, BEAM * BEAM), tpos, axis=1).astype(jnp.int32)
        new_scores = jnp.take_along_axis(
            total.reshape(B, BEAM * BEAM), tpos, axis=1).reshape(-1, 1)
        flat_beam = (beam_idx + jnp.arange(B)[:, None] * BEAM).reshape(-1)
        prefixes = jnp.concatenate(
            [prefixes[flat_beam], token.reshape(-1, 1)], axis=1)
        scores = new_scores
        pad_profile = jnp.maximum(
            pad_profile[flat_beam], (token.reshape(-1) == EOS).astype(jnp.int32))

    return prefixes, scores.reshape(B, BEAM)


# speedup vs baseline: 3.0483x; 1.0559x over previous
import functools

import jax
import jax.numpy as jnp
from jax.experimental import pallas as pl

V = 100000
D = 512
B = 16
BEAM = 4
STEPS = 6
PAD, BOS, EOS = 0, 1, 2
LEN_PEN = 0.6
NEG_INF = -1e9

VC = 10000
NC = V // VC
LANES = 128


def _chunk_kernel(iota_ref, hT_ref, emb_ref, m_ref, s_ref, v_ref, i_ref,
                  *, n_rows):
    nsub = LANES // n_rows
    sub = VC // nsub
    tiles = [
        jax.lax.dot_general(
            emb_ref[s * sub:(s + 1) * sub, :], hT_ref[...],
            dimension_numbers=(((1,), (0,)), ((), ())),
            preferred_element_type=jnp.float32)
        for s in range(nsub)
    ]
    logits = jnp.concatenate(tiles, axis=1)

    m = jnp.max(logits, axis=0)
    s_ref[0, 0, :] = jnp.sum(jnp.exp(logits - m[None, :]), axis=0)
    m_ref[0, 0, :] = m

    row = iota_ref[...]
    vals = logits
    mt = m
    for t in range(BEAM):
        if t:
            mt = jnp.max(vals, axis=0)
        ki = jnp.where(vals == mt[None, :], row, sub)
        at = jnp.min(ki, axis=0)
        v_ref[0, t, :] = mt
        i_ref[0, t, :] = at
        if t + 1 < BEAM:
            vals = jnp.where(ki == at[None, :], NEG_INF, vals)


def _stream_pass(hT, emb_table, n_rows):
    nsub = LANES // n_rows
    sub = VC // nsub
    iota = jnp.broadcast_to(
        jnp.arange(sub, dtype=jnp.int32)[:, None], (sub, LANES))
    kern = functools.partial(_chunk_kernel, n_rows=n_rows)
    return pl.pallas_call(
        kern,
        grid=(NC,),
        in_specs=[
            pl.BlockSpec((sub, LANES), lambda j: (0, 0)),
            pl.BlockSpec((D, n_rows), lambda j: (0, 0)),
            pl.BlockSpec((VC, D), lambda j: (j, 0)),
        ],
        out_specs=[
            pl.BlockSpec((1, 1, LANES), lambda j: (j, 0, 0)),
            pl.BlockSpec((1, 1, LANES), lambda j: (j, 0, 0)),
            pl.BlockSpec((1, BEAM, LANES), lambda j: (j, 0, 0)),
            pl.BlockSpec((1, BEAM, LANES), lambda j: (j, 0, 0)),
        ],
        out_shape=[
            jax.ShapeDtypeStruct((NC, 1, LANES), jnp.float32),
            jax.ShapeDtypeStruct((NC, 1, LANES), jnp.float32),
            jax.ShapeDtypeStruct((NC, BEAM, LANES), jnp.float32),
            jax.ShapeDtypeStruct((NC, BEAM, LANES), jnp.int32),
        ],
    )(iota, hT, emb_table)


def _merge_stats(m_c, s_c, v_c, i_c, n_rows):
    nsub = LANES // n_rows
    sub = VC // nsub
    m_c = m_c.reshape(NC, 1, nsub, n_rows)[:, 0]
    s_c = s_c.reshape(NC, 1, nsub, n_rows)[:, 0]
    m2 = m_c.reshape(NC * nsub, n_rows)
    s2 = s_c.reshape(NC * nsub, n_rows)
    mx = jnp.max(m2, axis=0)
    lse = mx + jnp.log(jnp.sum(s2 * jnp.exp(m2 - mx[None, :]), axis=0))
    v4 = v_c.reshape(NC, BEAM, nsub, n_rows)
    i4 = i_c.reshape(NC, BEAM, nsub, n_rows)
    chunk_base = (jnp.arange(NC, dtype=jnp.int32) * VC)[:, None, None, None]
    sub_base = (jnp.arange(nsub, dtype=jnp.int32) * sub)[None, None, :, None]
    g4 = chunk_base + sub_base + i4
    vals = v4.transpose(0, 2, 1, 3).reshape(NC * nsub * BEAM, n_rows).T
    idx = g4.transpose(0, 2, 1, 3).reshape(NC * nsub * BEAM, n_rows).T
    top_v, pos = jax.lax.top_k(vals, BEAM)
    top_i = jnp.take_along_axis(idx, pos, axis=1)
    return top_v, top_i, lse


def kernel(decoder_input_ids, encoder_hidden_states, encoder_input_mask,
           emb_table, pos_emb, W_dec, W_enc):
    mask = encoder_input_mask
    enc_ctx = (encoder_hidden_states * mask[:, :, None]).sum(axis=1) / \
        jnp.maximum(mask.sum(axis=1, keepdims=True), 1.0)

    tok0 = decoder_input_ids[:, 0]
    h = jnp.take(emb_table, tok0, axis=0) + pos_emb[0][None, :]
    h = jnp.tanh(h @ W_dec + enc_ctx @ W_enc)
    top_v, top_i, lse = _merge_stats(*_stream_pass(h.T, emb_table, B), B)
    scores = (top_v - lse[:, None]).reshape(-1, 1)
    prefixes = jnp.concatenate(
        [jnp.repeat(decoder_input_ids, BEAM, axis=0),
         top_i.reshape(-1, 1).astype(jnp.int32)], axis=1)
    ctx_rep = jnp.repeat(enc_ctx, BEAM, axis=0)
    ctx_enc = ctx_rep @ W_enc
    pad_profile = jnp.zeros((B * BEAM,), dtype=jnp.int32)
    n = B * BEAM

    for i in range(1, STEPS + 1):
        tok = prefixes[:, -1]
        h = jnp.take(emb_table, tok, axis=0) + pos_emb[i][None, :]
        h = jnp.tanh(h @ W_dec + ctx_enc)
        top_v, top_i, lse = _merge_stats(*_stream_pass(h.T, emb_table, n), n)
        lp_cand = top_v - lse[:, None]
        finished = pad_profile > 0
        pad_lp = jnp.where(jnp.arange(BEAM) == 0, 0.0, NEG_INF)
        lp_cand = jnp.where(finished[:, None], pad_lp[None, :], lp_cand)
        tok_cand = jnp.where(finished[:, None], PAD, top_i)

        total = scores + lp_cand
        length = prefixes.shape[1]
        penalty = ((5.0 + length) / 6.0) ** LEN_PEN
        cand = (total / penalty).reshape(B, BEAM * BEAM)
        _, tpos = jax.lax.top_k(cand, BEAM)
        beam_idx = tpos // BEAM
        token = jnp.take_along_axis(
            tok_cand.reshape(B, BEAM * BEAM), tpos, axis=1).astype(jnp.int32)
        new_scores = jnp.take_along_axis(
            total.reshape(B, BEAM * BEAM), tpos, axis=1).reshape(-1, 1)
        flat_beam = (beam_idx + jnp.arange(B)[:, None] * BEAM).reshape(-1)
        prefixes = jnp.concatenate(
            [prefixes[flat_beam], token.reshape(-1, 1)], axis=1)
        scores = new_scores
        pad_profile = jnp.maximum(
            pad_profile[flat_beam], (token.reshape(-1) == EOS).astype(jnp.int32))

    return prefixes, scores.reshape(B, BEAM)
